# baseline (device time: 98200 ns/iter reference)
import jax
import jax.numpy as jnp
from jax import lax
from jax.experimental import pallas as pl
from jax.experimental.pallas import tpu as pltpu

N_DEV = 8
CW, CCW = 0, 1
SUB = 2
A_STAGES = 4
B_STAGES = 3
A_BASE, B_BASE = 0, A_STAGES + 1
N_SLOTS = A_STAGES + B_STAGES + 2


def kernel(x, dy):
    m, d = x.shape
    _, f = dy.shape
    d_per = d // N_DEV
    f_half = f // 2
    d_sub = d_per // SUB

    def body(x_ref, dy_ref, out_ref, comm_ref, send_sems, recv_sems):
        my = lax.axis_index("i")

        idx = lax.broadcasted_iota(jnp.int32, (1, N_DEV), 1)
        ring_arr = jnp.where(idx < 4, idx, 11 - idx)
        pos = jnp.sum(jnp.where(ring_arr == my, idx, 0))

        def ring_at(k):
            return jnp.sum(jnp.where(idx == (k % N_DEV), ring_arr, 0))

        right = ring_at(pos + 1)
        left = ring_at(pos - 1)

        def contrib(c, half):
            xs = x_ref[:, pl.ds(c * d_per, d_per)]
            ys = dy_ref[:, pl.ds(half * f_half, f_half)]
            return lax.dot_general(
                xs, ys,
                dimension_numbers=(((0,), (0,)), ((), ())),
                preferred_element_type=jnp.float32,
            )

        def sig(link):
            return 1 if link == CW else -1

        def rdma(kind, link, s, j):
            base, semoff = (A_BASE, 0) if kind == "A" else (B_BASE, A_STAGES)
            tgt = right if link == CW else left
            return pltpu.make_async_remote_copy(
                src_ref=comm_ref.at[link, base + s, pl.ds(j * d_sub, d_sub), :],
                dst_ref=comm_ref.at[
                    link, base + s + 1, pl.ds(j * d_sub, d_sub), :
                ],
                send_sem=send_sems.at[link, semoff + s, j],
                recv_sem=recv_sems.at[link, semoff + s, j],
                device_id=(tgt,),
                device_id_type=pl.DeviceIdType.MESH,
            )

        live = {}

        def start_stage(kind, link, s):
            for j in range(SUB):
                r = rdma(kind, link, s, j)
                r.start()
                live[(kind, link, s, j)] = r

        def process(kind, link, s, merge_val):
            base = A_BASE if kind == "A" else B_BASE
            for j in range(SUB):
                live.pop((kind, link, s, j)).wait()
                rows = pl.ds(j * d_sub, d_sub)
                acc = (
                    comm_ref[link, base + s + 1, rows, :]
                    + merge_val[j * d_sub : (j + 1) * d_sub, :]
                )
                comm_ref[link, base + s + 1, rows, :] = acc
                r = rdma(kind, link, s + 1, j)
                r.start()
                live[(kind, link, s + 1, j)] = r

        barrier_sem = pltpu.get_barrier_semaphore()
        for nbr in (left, right):
            pl.semaphore_signal(
                barrier_sem, inc=1,
                device_id=(nbr,), device_id_type=pl.DeviceIdType.MESH,
            )
        for link in (CW, CCW):
            comm_ref[link, A_BASE, :, :] = contrib(
                ring_at(pos + sig(link) * 4), link
            )
            comm_ref[link, B_BASE, :, :] = contrib(
                ring_at(pos + sig(link) * 3), 1 - link
            )
        pl.semaphore_wait(barrier_sem, 2)

        for link in (CW, CCW):
            start_stage("A", link, 0)
        for link in (CW, CCW):
            start_stage("B", link, 0)

        mA = {
            (link, s): contrib(ring_at(pos + sig(link) * (3 - s)), link)
            for s in range(A_STAGES - 1)
            for link in (CW, CCW)
        }
        mB = {
            (link, s): contrib(ring_at(pos + sig(link) * (2 - s)), 1 - link)
            for s in range(B_STAGES - 1)
            for link in (CW, CCW)
        }

        process("A", CW, 0, mA[(CW, 0)])
        process("A", CCW, 0, mA[(CCW, 0)])
        process("B", CW, 0, mB[(CW, 0)])
        process("B", CCW, 0, mB[(CCW, 0)])
        process("A", CW, 1, mA[(CW, 1)])
        process("A", CCW, 1, mA[(CCW, 1)])
        process("B", CW, 1, mB[(CW, 1)])
        process("B", CCW, 1, mB[(CCW, 1)])
        process("A", CW, 2, mA[(CW, 2)])
        process("A", CCW, 2, mA[(CCW, 2)])

        fin = {half: contrib(my, half) for half in (0, 1)}

        for link in (CW, CCW):
            half = 1 - link
            for j in range(SUB):
                live.pop(("B", link, B_STAGES - 1, j)).wait()
                rows = pl.ds(j * d_sub, d_sub)
                out_ref[rows, pl.ds(half * f_half, f_half)] = (
                    comm_ref[link, B_BASE + B_STAGES, rows, :]
                    + fin[half][j * d_sub : (j + 1) * d_sub, :]
                )
        for link in (CW, CCW):
            half = link
            for j in range(SUB):
                live.pop(("A", link, A_STAGES - 1, j)).wait()
                rows = pl.ds(j * d_sub, d_sub)
                cols = pl.ds(half * f_half, f_half)
                out_ref[rows, cols] = (
                    out_ref[rows, cols]
                    + comm_ref[link, A_BASE + A_STAGES, rows, :]
                )

    return pl.pallas_call(
        body,
        out_shape=jax.ShapeDtypeStruct((d_per, f), jnp.float32),
        in_specs=[
            pl.BlockSpec(memory_space=pltpu.VMEM),
            pl.BlockSpec(memory_space=pltpu.VMEM),
        ],
        out_specs=pl.BlockSpec(memory_space=pltpu.VMEM),
        scratch_shapes=[
            pltpu.VMEM((2, N_SLOTS, d_per, f_half), jnp.float32),
            pltpu.SemaphoreType.DMA((2, A_STAGES + B_STAGES, SUB)),
            pltpu.SemaphoreType.DMA((2, A_STAGES + B_STAGES, SUB)),
        ],
        compiler_params=pltpu.CompilerParams(
            collective_id=0,
            vmem_limit_bytes=100 * 1024 * 1024,
        ),
    )(x, dy)


# device time: 88529 ns/iter; 1.1092x vs baseline; 1.1092x over previous
import jax
import jax.numpy as jnp
from jax import lax
from jax.experimental import pallas as pl
from jax.experimental.pallas import tpu as pltpu

N_DEV = 8
CW, CCW = 0, 1
SUB = 2
STAGES = 3
SHC_OUT, SHC_IN = 0, 1


def kernel(x, dy):
    m, d = x.shape
    _, f = dy.shape
    d_per = d // N_DEV
    d_sub = d_per // SUB

    def body(x_ref, dy_ref, out_ref, comm_ref, shc_ref,
             send_sems, recv_sems, shc_send_sems, shc_recv_sems):
        my = lax.axis_index("i")

        idx = lax.broadcasted_iota(jnp.int32, (1, N_DEV), 1)
        ring_arr = jnp.where(idx < 4, idx, 11 - idx)
        pos = jnp.sum(jnp.where(ring_arr == my, idx, 0))

        def ring_at(k):
            return jnp.sum(jnp.where(idx == (k % N_DEV), ring_arr, 0))

        right = ring_at(pos + 1)
        left = ring_at(pos - 1)
        even = pos % 2 == 0
        partner = jnp.where(even, ring_at(pos + 3), ring_at(pos - 3))

        def contrib(c):
            xs = x_ref[:, pl.ds(c * d_per, d_per)]
            return lax.dot_general(
                xs, dy_ref[:, :],
                dimension_numbers=(((0,), (0,)), ((), ())),
                preferred_element_type=jnp.float32,
            )

        def sig(link):
            return 1 if link == CW else -1

        def rdma(link, s, j):
            tgt = right if link == CW else left
            return pltpu.make_async_remote_copy(
                src_ref=comm_ref.at[link, s, pl.ds(j * d_sub, d_sub), :],
                dst_ref=comm_ref.at[link, s + 1, pl.ds(j * d_sub, d_sub), :],
                send_sem=send_sems.at[link, s, j],
                recv_sem=recv_sems.at[link, s, j],
                device_id=(tgt,),
                device_id_type=pl.DeviceIdType.MESH,
            )

        def shc_rdma(j):
            return pltpu.make_async_remote_copy(
                src_ref=shc_ref.at[SHC_OUT, pl.ds(j * d_sub, d_sub), :],
                dst_ref=shc_ref.at[SHC_IN, pl.ds(j * d_sub, d_sub), :],
                send_sem=shc_send_sems.at[j],
                recv_sem=shc_recv_sems.at[j],
                device_id=(partner,),
                device_id_type=pl.DeviceIdType.MESH,
            )

        barrier_sem = pltpu.get_barrier_semaphore()
        for nbr in (left, right, partner):
            pl.semaphore_signal(
                barrier_sem, inc=1,
                device_id=(nbr,), device_id_type=pl.DeviceIdType.MESH,
            )
        comm_ref[CW, 0, :, :] = contrib(ring_at(pos + 3))
        comm_ref[CCW, 0, :, :] = contrib(ring_at(pos - 3))
        shc_ref[SHC_OUT, :, :] = contrib(ring_at(pos + 4))
        pl.semaphore_wait(barrier_sem, 3)

        live = {}
        for link in (CW, CCW):
            for j in range(SUB):
                r = rdma(link, 0, j)
                r.start()
                live[(link, 0, j)] = r
        shc_live = []
        for j in range(SUB):
            r = shc_rdma(j)
            r.start()
            shc_live.append(r)

        merge = {
            (link, s): contrib(ring_at(pos + sig(link) * (2 - s)))
            for s in range(STAGES - 1)
            for link in (CW, CCW)
        }
        own = contrib(my)

        for link in (CW, CCW):
            for j in range(SUB):
                live.pop((link, 0, j)).wait()
                rows = pl.ds(j * d_sub, d_sub)
                comm_ref[link, 1, rows, :] = (
                    comm_ref[link, 1, rows, :]
                    + merge[(link, 0)][j * d_sub : (j + 1) * d_sub, :]
                )
                r = rdma(link, 1, j)
                r.start()
                live[(link, 1, j)] = r

        for r in shc_live:
            r.wait()

        for link in (CW, CCW):
            shc_here = jnp.logical_not(even) if link == CW else even
            for j in range(SUB):
                live.pop((link, 1, j)).wait()
                rows = pl.ds(j * d_sub, d_sub)
                comm_ref[link, 2, rows, :] = (
                    comm_ref[link, 2, rows, :]
                    + merge[(link, 1)][j * d_sub : (j + 1) * d_sub, :]
                )

                @pl.when(shc_here)
                def _():
                    comm_ref[link, 2, rows, :] = (
                        comm_ref[link, 2, rows, :] + shc_ref[SHC_IN, rows, :]
                    )

                r = rdma(link, 2, j)
                r.start()
                live[(link, 2, j)] = r

        for j in range(SUB):
            live.pop((CW, 2, j)).wait()
            live.pop((CCW, 2, j)).wait()
            rows = pl.ds(j * d_sub, d_sub)
            out_ref[rows, :] = (
                comm_ref[CW, STAGES, rows, :]
                + comm_ref[CCW, STAGES, rows, :]
                + own[j * d_sub : (j + 1) * d_sub, :]
            )

    return pl.pallas_call(
        body,
        out_shape=jax.ShapeDtypeStruct((d_per, f), jnp.float32),
        in_specs=[
            pl.BlockSpec(memory_space=pltpu.VMEM),
            pl.BlockSpec(memory_space=pltpu.VMEM),
        ],
        out_specs=pl.BlockSpec(memory_space=pltpu.VMEM),
        scratch_shapes=[
            pltpu.VMEM((2, STAGES + 1, d_per, f), jnp.float32),
            pltpu.VMEM((2, d_per, f), jnp.float32),
            pltpu.SemaphoreType.DMA((2, STAGES, SUB)),
            pltpu.SemaphoreType.DMA((2, STAGES, SUB)),
            pltpu.SemaphoreType.DMA((SUB,)),
            pltpu.SemaphoreType.DMA((SUB,)),
        ],
        compiler_params=pltpu.CompilerParams(
            collective_id=0,
            vmem_limit_bytes=100 * 1024 * 1024,
        ),
    )(x, dy)


# device time: 86616 ns/iter; 1.1337x vs baseline; 1.0221x over previous
import jax
import jax.numpy as jnp
from jax import lax
from jax.experimental import pallas as pl
from jax.experimental.pallas import tpu as pltpu

N_DEV = 8
CW, CCW = 0, 1
SUB = 2
STAGES = 3
SHC_OUT, SHC_IN = 0, 1


def kernel(x, dy):
    m, d = x.shape
    _, f = dy.shape
    d_per = d // N_DEV
    d_sub = d_per // SUB

    def body(x_ref, dy_ref, out_ref, comm_ref, shc_ref,
             send_sems, recv_sems, shc_send_sems, shc_recv_sems):
        my = lax.axis_index("i")

        idx = lax.broadcasted_iota(jnp.int32, (1, N_DEV), 1)
        ring_arr = jnp.where(idx < 4, idx, 11 - idx)
        pos = jnp.sum(jnp.where(ring_arr == my, idx, 0))

        def ring_at(k):
            return jnp.sum(jnp.where(idx == (k % N_DEV), ring_arr, 0))

        right = ring_at(pos + 1)
        left = ring_at(pos - 1)
        even = pos % 2 == 0
        partner = jnp.where(even, ring_at(pos + 3), ring_at(pos - 3))

        def contrib(c):
            xs = x_ref[:, pl.ds(c * d_per, d_per)]
            return lax.dot_general(
                xs, dy_ref[:, :],
                dimension_numbers=(((0,), (0,)), ((), ())),
                preferred_element_type=jnp.float32,
            )

        def sig(link):
            return 1 if link == CW else -1

        def rdma(link, s, j):
            tgt = right if link == CW else left
            return pltpu.make_async_remote_copy(
                src_ref=comm_ref.at[link, s, pl.ds(j * d_sub, d_sub), :],
                dst_ref=comm_ref.at[link, s + 1, pl.ds(j * d_sub, d_sub), :],
                send_sem=send_sems.at[link, s, j],
                recv_sem=recv_sems.at[link, s, j],
                device_id=(tgt,),
                device_id_type=pl.DeviceIdType.MESH,
            )

        def shc_rdma(j):
            return pltpu.make_async_remote_copy(
                src_ref=shc_ref.at[SHC_OUT, pl.ds(j * d_sub, d_sub), :],
                dst_ref=shc_ref.at[SHC_IN, pl.ds(j * d_sub, d_sub), :],
                send_sem=shc_send_sems.at[j],
                recv_sem=shc_recv_sems.at[j],
                device_id=(partner,),
                device_id_type=pl.DeviceIdType.MESH,
            )

        barrier_sem = pltpu.get_barrier_semaphore()
        for nbr in (left, right, partner):
            pl.semaphore_signal(
                barrier_sem, inc=1,
                device_id=(nbr,), device_id_type=pl.DeviceIdType.MESH,
            )
        comm_ref[CW, 0, :, :] = contrib(ring_at(pos + 3))
        pl.semaphore_wait(barrier_sem, 3)

        live = {}
        for j in range(SUB):
            r = rdma(CW, 0, j)
            r.start()
            live[(CW, 0, j)] = r
        comm_ref[CCW, 0, :, :] = contrib(ring_at(pos - 3))
        for j in range(SUB):
            r = rdma(CCW, 0, j)
            r.start()
            live[(CCW, 0, j)] = r
        shc_ref[SHC_OUT, :, :] = contrib(ring_at(pos + 4))
        shc_live = []
        for j in range(SUB):
            r = shc_rdma(j)
            r.start()
            shc_live.append(r)

        merge = {
            (link, s): contrib(ring_at(pos + sig(link) * (2 - s)))
            for s in range(STAGES - 1)
            for link in (CW, CCW)
        }
        own = contrib(my)

        for link in (CW, CCW):
            for j in range(SUB):
                live.pop((link, 0, j)).wait()
                rows = pl.ds(j * d_sub, d_sub)
                comm_ref[link, 1, rows, :] = (
                    comm_ref[link, 1, rows, :]
                    + merge[(link, 0)][j * d_sub : (j + 1) * d_sub, :]
                )
                r = rdma(link, 1, j)
                r.start()
                live[(link, 1, j)] = r

        for r in shc_live:
            r.wait()

        for link in (CW, CCW):
            shc_here = jnp.logical_not(even) if link == CW else even
            for j in range(SUB):
                live.pop((link, 1, j)).wait()
                rows = pl.ds(j * d_sub, d_sub)
                comm_ref[link, 2, rows, :] = (
                    comm_ref[link, 2, rows, :]
                    + merge[(link, 1)][j * d_sub : (j + 1) * d_sub, :]
                )

                @pl.when(shc_here)
                def _():
                    comm_ref[link, 2, rows, :] = (
                        comm_ref[link, 2, rows, :] + shc_ref[SHC_IN, rows, :]
                    )

                r = rdma(link, 2, j)
                r.start()
                live[(link, 2, j)] = r

        for j in range(SUB):
            live.pop((CW, 2, j)).wait()
            live.pop((CCW, 2, j)).wait()
            rows = pl.ds(j * d_sub, d_sub)
            out_ref[rows, :] = (
                comm_ref[CW, STAGES, rows, :]
                + comm_ref[CCW, STAGES, rows, :]
                + own[j * d_sub : (j + 1) * d_sub, :]
            )

    return pl.pallas_call(
        body,
        out_shape=jax.ShapeDtypeStruct((d_per, f), jnp.float32),
        in_specs=[
            pl.BlockSpec(memory_space=pltpu.VMEM),
            pl.BlockSpec(memory_space=pltpu.VMEM),
        ],
        out_specs=pl.BlockSpec(memory_space=pltpu.VMEM),
        scratch_shapes=[
            pltpu.VMEM((2, STAGES + 1, d_per, f), jnp.float32),
            pltpu.VMEM((2, d_per, f), jnp.float32),
            pltpu.SemaphoreType.DMA((2, STAGES, SUB)),
            pltpu.SemaphoreType.DMA((2, STAGES, SUB)),
            pltpu.SemaphoreType.DMA((SUB,)),
            pltpu.SemaphoreType.DMA((SUB,)),
        ],
        compiler_params=pltpu.CompilerParams(
            collective_id=0,
            vmem_limit_bytes=100 * 1024 * 1024,
        ),
    )(x, dy)


# device time: 86613 ns/iter; 1.1338x vs baseline; 1.0000x over previous
import jax
import jax.numpy as jnp
from jax import lax
from jax.experimental import pallas as pl
from jax.experimental.pallas import tpu as pltpu

N_DEV = 8
CW, CCW = 0, 1
SUB = 2
STAGES = 3
SHC_OUT, SHC_IN = 0, 1


def kernel(x, dy):
    m, d = x.shape
    _, f = dy.shape
    d_per = d // N_DEV
    d_sub = d_per // SUB

    def body(x_ref, dy_ref, out_ref, comm_ref, shc_ref,
             send_sems, recv_sems, shc_send_sems, shc_recv_sems):
        my = lax.axis_index("i")

        idx = lax.broadcasted_iota(jnp.int32, (1, N_DEV), 1)
        ring_arr = jnp.where(idx < 4, idx, 11 - idx)
        pos = jnp.sum(jnp.where(ring_arr == my, idx, 0))

        def ring_at(k):
            return jnp.sum(jnp.where(idx == (k % N_DEV), ring_arr, 0))

        right = ring_at(pos + 1)
        left = ring_at(pos - 1)
        even = pos % 2 == 0
        partner = jnp.where(even, ring_at(pos + 3), ring_at(pos - 3))

        def contrib(c):
            xs = x_ref[:, pl.ds(c * d_per, d_per)]
            return lax.dot_general(
                xs, dy_ref[:, :],
                dimension_numbers=(((0,), (0,)), ((), ())),
                preferred_element_type=jnp.float32,
            )

        def sig(link):
            return 1 if link == CW else -1

        def rdma(link, s, j):
            tgt = right if link == CW else left
            return pltpu.make_async_remote_copy(
                src_ref=comm_ref.at[link, s, pl.ds(j * d_sub, d_sub), :],
                dst_ref=comm_ref.at[link, s + 1, pl.ds(j * d_sub, d_sub), :],
                send_sem=send_sems.at[link, s, j],
                recv_sem=recv_sems.at[link, s, j],
                device_id=(tgt,),
                device_id_type=pl.DeviceIdType.MESH,
            )

        def shc_rdma(j):
            return pltpu.make_async_remote_copy(
                src_ref=shc_ref.at[SHC_OUT, pl.ds(j * d_sub, d_sub), :],
                dst_ref=shc_ref.at[SHC_IN, pl.ds(j * d_sub, d_sub), :],
                send_sem=shc_send_sems.at[j],
                recv_sem=shc_recv_sems.at[j],
                device_id=(partner,),
                device_id_type=pl.DeviceIdType.MESH,
            )

        barrier_sem = pltpu.get_barrier_semaphore()
        for nbr in (left, right, partner):
            pl.semaphore_signal(
                barrier_sem, inc=1,
                device_id=(nbr,), device_id_type=pl.DeviceIdType.MESH,
            )
        comm_ref[CW, 0, :, :] = contrib(ring_at(pos + 3))
        pl.semaphore_wait(barrier_sem, 3)

        live = {}
        for j in range(SUB):
            r = rdma(CW, 0, j)
            r.start()
            live[(CW, 0, j)] = r
        comm_ref[CCW, 0, :, :] = contrib(ring_at(pos - 3))
        for j in range(SUB):
            r = rdma(CCW, 0, j)
            r.start()
            live[(CCW, 0, j)] = r
        shc_ref[SHC_OUT, :, :] = contrib(ring_at(pos + 4))
        shc_live = []
        for j in range(SUB):
            r = shc_rdma(j)
            r.start()
            shc_live.append(r)

        merge = {
            (link, s): contrib(ring_at(pos + sig(link) * (2 - s)))
            for s in range(STAGES - 1)
            for link in (CW, CCW)
        }
        own = contrib(my)

        for link in (CW, CCW):
            for j in range(SUB):
                live.pop((link, 0, j)).wait()
                rows = pl.ds(j * d_sub, d_sub)
                comm_ref[link, 1, rows, :] = (
                    comm_ref[link, 1, rows, :]
                    + merge[(link, 0)][j * d_sub : (j + 1) * d_sub, :]
                )
                r = rdma(link, 1, j)
                r.start()
                live[(link, 1, j)] = r

        for r in shc_live:
            r.wait()

        for link in (CW, CCW):
            shc_here = jnp.logical_not(even) if link == CW else even
            for j in range(SUB):
                live.pop((link, 1, j)).wait()
                rows = pl.ds(j * d_sub, d_sub)
                m = merge[(link, 1)][j * d_sub : (j + 1) * d_sub, :]

                @pl.when(shc_here)
                def _():
                    comm_ref[link, 2, rows, :] = (
                        comm_ref[link, 2, rows, :]
                        + m
                        + shc_ref[SHC_IN, rows, :]
                    )

                @pl.when(jnp.logical_not(shc_here))
                def _():
                    comm_ref[link, 2, rows, :] = comm_ref[link, 2, rows, :] + m

                r = rdma(link, 2, j)
                r.start()
                live[(link, 2, j)] = r

        for j in range(SUB):
            live.pop((CW, 2, j)).wait()
            live.pop((CCW, 2, j)).wait()
            rows = pl.ds(j * d_sub, d_sub)
            out_ref[rows, :] = (
                comm_ref[CW, STAGES, rows, :]
                + comm_ref[CCW, STAGES, rows, :]
                + own[j * d_sub : (j + 1) * d_sub, :]
            )

    return pl.pallas_call(
        body,
        out_shape=jax.ShapeDtypeStruct((d_per, f), jnp.float32),
        in_specs=[
            pl.BlockSpec(memory_space=pltpu.VMEM),
            pl.BlockSpec(memory_space=pltpu.VMEM),
        ],
        out_specs=pl.BlockSpec(memory_space=pltpu.VMEM),
        scratch_shapes=[
            pltpu.VMEM((2, STAGES + 1, d_per, f), jnp.float32),
            pltpu.VMEM((2, d_per, f), jnp.float32),
            pltpu.SemaphoreType.DMA((2, STAGES, SUB)),
            pltpu.SemaphoreType.DMA((2, STAGES, SUB)),
            pltpu.SemaphoreType.DMA((SUB,)),
            pltpu.SemaphoreType.DMA((SUB,)),
        ],
        compiler_params=pltpu.CompilerParams(
            collective_id=0,
            vmem_limit_bytes=100 * 1024 * 1024,
        ),
    )(x, dy)
